# baseline (speedup 1.0000x reference)
import jax
import jax.numpy as jnp
from jax.experimental import pallas as pl
from jax.experimental.pallas import tpu as pltpu

_N = 10000
_D = 128
_BLK = 2048

_DN0 = (((0,), (0,)), ((), ()))


def _fused_mlp_kernel(x_ref, enW0_ref, enb0_ref, enW1_ref, enb1_ref,
                      enW2_ref, enb2_ref, en_g_ref, en_b_ref,
                      dW0_ref, db0_ref, dW1_ref, db1_ref, dW2_ref, db2_ref,
                      y_ref):
    f32 = jnp.float32

    def tdot(w_ref, h):
        return jnp.dot(w_ref[...], h, preferred_element_type=f32)

    h = jnp.maximum(tdot(enW0_ref, x_ref[...]) + enb0_ref[...], 0.0)
    h = jnp.maximum(tdot(enW1_ref, h) + enb1_ref[...], 0.0)
    h = tdot(enW2_ref, h) + enb2_ref[...]
    m = jnp.mean(h, axis=0, keepdims=True)
    c = h - m
    v = jnp.mean(c * c, axis=0, keepdims=True)
    h = c * jax.lax.rsqrt(v + 1e-5) * en_g_ref[...] + en_b_ref[...]
    h = jnp.maximum(tdot(dW0_ref, h) + db0_ref[...], 0.0)
    h = jnp.maximum(tdot(dW1_ref, h) + db1_ref[...], 0.0)
    y_ref[...] = jax.lax.dot_general(h, dW2_ref[...], _DN0,
                                     preferred_element_type=f32) + db2_ref[...]


@jax.jit
def kernel(x, edge_index, edge_features, params):
    del edge_index, edge_features
    p = params
    nout = p['dW2'].shape[1]

    def col(v):
        return v.reshape(v.shape[0], 1)

    operands = (x.T,
                p['enW0'].T, col(p['enb0']),
                p['enW1'].T, col(p['enb1']),
                p['enW2'].T, col(p['enb2']),
                col(p['en_g']), col(p['en_b']),
                p['dW0'].T, col(p['db0']),
                p['dW1'].T, col(p['db1']),
                p['dW2'], p['db2'].reshape(1, nout))

    grid = (pl.cdiv(_N, _BLK),)
    xt_spec = pl.BlockSpec((x.shape[1], _BLK), lambda i: (0, i))
    out_spec = pl.BlockSpec((_BLK, nout), lambda i: (i, 0))

    def full(a):
        return pl.BlockSpec(a.shape, lambda i: (0, 0))

    in_specs = [xt_spec] + [full(a) for a in operands[1:]]

    return pl.pallas_call(
        _fused_mlp_kernel,
        grid=grid,
        in_specs=in_specs,
        out_specs=out_spec,
        out_shape=jax.ShapeDtypeStruct((_N, nout), jnp.float32),
        compiler_params=pltpu.CompilerParams(
            dimension_semantics=("parallel",),
        ),
    )(*operands)


# speedup vs baseline: 1.2949x; 1.2949x over previous
import jax
import jax.numpy as jnp
from jax.experimental import pallas as pl
from jax.experimental.pallas import tpu as pltpu

_N = 10000
_D = 128
_BLK = 2048

_DN0 = (((0,), (0,)), ((), ()))


def _fused_mlp_kernel(x_ref, enW0_ref, enb0_ref, enW1_ref, enb1_ref,
                      enW2_ref, enb2_ref, en_g_ref, en_b_ref,
                      dW0_ref, db0_ref, dW1_ref, db1_ref, dW2_ref, db2_ref,
                      y_ref):
    f32 = jnp.float32

    def tdot(w_ref, h):
        return jax.lax.dot_general(w_ref[...], h, _DN0,
                                   preferred_element_type=f32)

    h = jnp.maximum(tdot(enW0_ref, x_ref[...]) + enb0_ref[...], 0.0)
    h = jnp.maximum(tdot(enW1_ref, h) + enb1_ref[...], 0.0)
    h = tdot(enW2_ref, h) + enb2_ref[...]
    m = jnp.mean(h, axis=0, keepdims=True)
    c = h - m
    v = jnp.mean(c * c, axis=0, keepdims=True)
    h = c * jax.lax.rsqrt(v + 1e-5)
    dW0g = dW0_ref[...] * en_g_ref[...]
    db0f = jax.lax.dot_general(dW0_ref[...], en_b_ref[...],
                               (((0,), (0,)), ((), ())),
                               preferred_element_type=jnp.float32) + db0_ref[...]
    h = jnp.maximum(jax.lax.dot_general(dW0g, h, _DN0,
                                        preferred_element_type=jnp.float32)
                    + db0f, 0.0)
    h = jnp.maximum(tdot(dW1_ref, h) + db1_ref[...], 0.0)
    y_ref[...] = jax.lax.dot_general(h, dW2_ref[...], _DN0,
                                     preferred_element_type=f32) + db2_ref[...]


@jax.jit
def kernel(x, edge_index, edge_features, params):
    del edge_index, edge_features
    p = params
    nout = p['dW2'].shape[1]

    def col(v):
        return v.reshape(v.shape[0], 1)

    operands = (x.T,
                p['enW0'], col(p['enb0']),
                p['enW1'], col(p['enb1']),
                p['enW2'], col(p['enb2']),
                col(p['en_g']), col(p['en_b']),
                p['dW0'], col(p['db0']),
                p['dW1'], col(p['db1']),
                p['dW2'], p['db2'].reshape(1, nout))

    grid = (pl.cdiv(_N, _BLK),)
    xt_spec = pl.BlockSpec((x.shape[1], _BLK), lambda i: (0, i))
    out_spec = pl.BlockSpec((_BLK, nout), lambda i: (i, 0))

    def full(a):
        return pl.BlockSpec(a.shape, lambda i: (0, 0))

    in_specs = [xt_spec] + [full(a) for a in operands[1:]]

    return pl.pallas_call(
        _fused_mlp_kernel,
        grid=grid,
        in_specs=in_specs,
        out_specs=out_spec,
        out_shape=jax.ShapeDtypeStruct((_N, nout), jnp.float32),
        compiler_params=pltpu.CompilerParams(
            dimension_semantics=("parallel",),
        ),
    )(*operands)


# speedup vs baseline: 2.1897x; 1.6910x over previous
import functools

import jax
import jax.numpy as jnp
from jax.experimental import pallas as pl
from jax.experimental.pallas import tpu as pltpu

_N = 10000
_D = 128
_BLK = 2560


def _fused_mlp_kernel(x_ref, enW0_ref, enb0_ref, enW1_ref, enb1_ref,
                      enW2_ref, enb2_ref, en_g_ref, en_b_ref,
                      dW0_ref, db0_ref, dW1_ref, db1_ref, dW2_ref, db2_ref,
                      y_ref):
    f32 = jnp.float32
    h = jnp.maximum(
        jax.lax.dot_general(
            x_ref[...], enW0_ref[...], (((0,), (0,)), ((), ())),
            preferred_element_type=f32)
        + enb0_ref[...], 0.0)
    h = jnp.maximum(
        jnp.dot(h, enW1_ref[...], preferred_element_type=f32)
        + enb1_ref[...], 0.0)
    h = jnp.dot(h, enW2_ref[...], preferred_element_type=f32) + enb2_ref[...]
    m = jnp.mean(h, axis=-1, keepdims=True)
    c = h - m
    v = jnp.mean(c * c, axis=-1, keepdims=True)
    h = c * jax.lax.rsqrt(v + 1e-5) * en_g_ref[...] + en_b_ref[...]
    h = jnp.maximum(
        jnp.dot(h, dW0_ref[...], preferred_element_type=f32)
        + db0_ref[...], 0.0)
    h = jnp.maximum(
        jnp.dot(h, dW1_ref[...], preferred_element_type=f32)
        + db1_ref[...], 0.0)
    y_ref[...] = (jnp.dot(h, dW2_ref[...], preferred_element_type=f32)
                  + db2_ref[...])


@jax.jit
def kernel(x, edge_index, edge_features, params):
    del edge_index, edge_features
    p = params
    nout = p['dW2'].shape[1]

    def row(v):
        return v.reshape(1, v.shape[0])

    operands = (x.T,
                p['enW0'], row(p['enb0']),
                p['enW1'], row(p['enb1']),
                p['enW2'], row(p['enb2']),
                row(p['en_g']), row(p['en_b']),
                p['dW0'], row(p['db0']),
                p['dW1'], row(p['db1']),
                p['dW2'], row(p['db2']))

    grid = (pl.cdiv(_N, _BLK),)
    row_spec = pl.BlockSpec((x.shape[1], _BLK), lambda i: (0, i))
    out_spec = pl.BlockSpec((_BLK, nout), lambda i: (i, 0))

    def full(a):
        return pl.BlockSpec(a.shape, lambda i: (0, 0))

    in_specs = [row_spec] + [full(a) for a in operands[1:]]

    return pl.pallas_call(
        _fused_mlp_kernel,
        grid=grid,
        in_specs=in_specs,
        out_specs=out_spec,
        out_shape=jax.ShapeDtypeStruct((_N, nout), jnp.float32),
        compiler_params=pltpu.CompilerParams(
            dimension_semantics=("parallel",),
        ),
    )(*operands)


# speedup vs baseline: 2.2169x; 1.0124x over previous
import functools

import jax
import jax.numpy as jnp
from jax.experimental import pallas as pl
from jax.experimental.pallas import tpu as pltpu

_N = 10000
_D = 128
_BLK = 3456


def _fused_mlp_kernel(x_ref, enW0_ref, enb0_ref, enW1_ref, enb1_ref,
                      enW2_ref, enb2_ref, en_g_ref, en_b_ref,
                      dW0_ref, db0_ref, dW1_ref, db1_ref, dW2_ref, db2_ref,
                      y_ref):
    f32 = jnp.float32
    h = jnp.maximum(
        jax.lax.dot_general(
            x_ref[...], enW0_ref[...], (((0,), (0,)), ((), ())),
            preferred_element_type=f32)
        + enb0_ref[...], 0.0)
    h = jnp.maximum(
        jnp.dot(h, enW1_ref[...], preferred_element_type=f32)
        + enb1_ref[...], 0.0)
    h = jnp.dot(h, enW2_ref[...], preferred_element_type=f32) + enb2_ref[...]
    m = jnp.mean(h, axis=-1, keepdims=True)
    c = h - m
    v = jnp.mean(c * c, axis=-1, keepdims=True)
    h = c * jax.lax.rsqrt(v + 1e-5) * en_g_ref[...] + en_b_ref[...]
    h = jnp.maximum(
        jnp.dot(h, dW0_ref[...], preferred_element_type=f32)
        + db0_ref[...], 0.0)
    h = jnp.maximum(
        jnp.dot(h, dW1_ref[...], preferred_element_type=f32)
        + db1_ref[...], 0.0)
    y_ref[...] = (jnp.dot(h, dW2_ref[...], preferred_element_type=f32)
                  + db2_ref[...])


@jax.jit
def kernel(x, edge_index, edge_features, params):
    del edge_index, edge_features
    p = params
    nout = p['dW2'].shape[1]

    def row(v):
        return v.reshape(1, v.shape[0])

    operands = (x.T,
                p['enW0'], row(p['enb0']),
                p['enW1'], row(p['enb1']),
                p['enW2'], row(p['enb2']),
                row(p['en_g']), row(p['en_b']),
                p['dW0'], row(p['db0']),
                p['dW1'], row(p['db1']),
                p['dW2'], row(p['db2']))

    grid = (pl.cdiv(_N, _BLK),)
    row_spec = pl.BlockSpec((x.shape[1], _BLK), lambda i: (0, i))
    out_spec = pl.BlockSpec((_BLK, nout), lambda i: (i, 0))

    def full(a):
        return pl.BlockSpec(a.shape, lambda i: (0, 0))

    in_specs = [row_spec] + [full(a) for a in operands[1:]]

    return pl.pallas_call(
        _fused_mlp_kernel,
        grid=grid,
        in_specs=in_specs,
        out_specs=out_spec,
        out_shape=jax.ShapeDtypeStruct((_N, nout), jnp.float32),
        compiler_params=pltpu.CompilerParams(
            dimension_semantics=("parallel",),
        ),
    )(*operands)


# speedup vs baseline: 2.2666x; 1.0224x over previous
import functools

import jax
import jax.numpy as jnp
from jax.experimental import pallas as pl
from jax.experimental.pallas import tpu as pltpu

_N = 10000
_D = 128
_BLK = 5120


def _fused_mlp_kernel(x_ref, enW0_ref, enb0_ref, enW1_ref, enb1_ref,
                      enW2_ref, enb2_ref, en_g_ref, en_b_ref,
                      dW0_ref, db0_ref, dW1_ref, db1_ref, dW2_ref, db2_ref,
                      y_ref):
    f32 = jnp.float32
    h = jnp.maximum(
        jax.lax.dot_general(
            x_ref[...], enW0_ref[...], (((0,), (0,)), ((), ())),
            preferred_element_type=f32)
        + enb0_ref[...], 0.0)
    h = jnp.maximum(
        jnp.dot(h, enW1_ref[...], preferred_element_type=f32)
        + enb1_ref[...], 0.0)
    h = jnp.dot(h, enW2_ref[...], preferred_element_type=f32) + enb2_ref[...]
    m = jnp.mean(h, axis=-1, keepdims=True)
    c = h - m
    v = jnp.mean(c * c, axis=-1, keepdims=True)
    h = c * jax.lax.rsqrt(v + 1e-5) * en_g_ref[...] + en_b_ref[...]
    h = jnp.maximum(
        jnp.dot(h, dW0_ref[...], preferred_element_type=f32)
        + db0_ref[...], 0.0)
    h = jnp.maximum(
        jnp.dot(h, dW1_ref[...], preferred_element_type=f32)
        + db1_ref[...], 0.0)
    y_ref[...] = (jnp.dot(h, dW2_ref[...], preferred_element_type=f32)
                  + db2_ref[...])


@jax.jit
def kernel(x, edge_index, edge_features, params):
    del edge_index, edge_features
    p = params
    nout = p['dW2'].shape[1]

    def row(v):
        return v.reshape(1, v.shape[0])

    operands = (x.T,
                p['enW0'], row(p['enb0']),
                p['enW1'], row(p['enb1']),
                p['enW2'], row(p['enb2']),
                row(p['en_g']), row(p['en_b']),
                p['dW0'], row(p['db0']),
                p['dW1'], row(p['db1']),
                p['dW2'], row(p['db2']))

    grid = (pl.cdiv(_N, _BLK),)
    row_spec = pl.BlockSpec((x.shape[1], _BLK), lambda i: (0, i))
    out_spec = pl.BlockSpec((_BLK, nout), lambda i: (i, 0))

    def full(a):
        return pl.BlockSpec(a.shape, lambda i: (0, 0))

    in_specs = [row_spec] + [full(a) for a in operands[1:]]

    return pl.pallas_call(
        _fused_mlp_kernel,
        grid=grid,
        in_specs=in_specs,
        out_specs=out_spec,
        out_shape=jax.ShapeDtypeStruct((_N, nout), jnp.float32),
        compiler_params=pltpu.CompilerParams(
            dimension_semantics=("parallel",),
        ),
    )(*operands)
